# initial kernel scaffold (unmeasured)
import jax
import jax.numpy as jnp
from jax import lax
from jax.experimental import pallas as pl
from jax.experimental.pallas import tpu as pltpu


def kernel(
    x,
):
    def body(*refs):
        pass

    out_shape = jax.ShapeDtypeStruct(..., jnp.float32)
    return pl.pallas_call(body, out_shape=out_shape)(...)


# baseline (device time: 847336 ns/iter reference)
import jax
import jax.numpy as jnp
from jax import lax
from jax.experimental import pallas as pl
from jax.experimental.pallas import tpu as pltpu

M = 16384
N = 2048
HALF = N // 2
CHUNK = 2048
N_CHUNKS = M // CHUNK


def kernel(x):

    def body(x_ref, out_ref, recv_ref, va, vb, vo, dma_sems, send_sem, recv_sem):
        my_x = lax.axis_index("x")
        my_y = lax.axis_index("y")
        my_z = lax.axis_index("z")
        peer_z = 1 - my_z
        peer = (my_x, my_y, peer_z)

        barrier_sem = pltpu.get_barrier_semaphore()
        pl.semaphore_signal(
            barrier_sem, inc=1, device_id=peer,
            device_id_type=pl.DeviceIdType.MESH,
        )
        pl.semaphore_wait(barrier_sem, 1)

        rdma = pltpu.make_async_remote_copy(
            src_ref=x_ref.at[0, :, pl.ds(peer_z * HALF, HALF)],
            dst_ref=recv_ref,
            send_sem=send_sem,
            recv_sem=recv_sem,
            device_id=peer,
            device_id_type=pl.DeviceIdType.MESH,
        )
        rdma.start()
        rdma.wait()

        for i in range(N_CHUNKS):
            rows = pl.ds(i * CHUNK, CHUNK)
            cp_a = pltpu.make_async_copy(
                x_ref.at[0, rows, pl.ds(my_z * HALF, HALF)], va, dma_sems.at[0]
            )
            cp_b = pltpu.make_async_copy(recv_ref.at[rows, :], vb, dma_sems.at[1])
            cp_a.start()
            cp_b.start()
            cp_a.wait()
            cp_b.wait()
            vo[...] = va[...] + vb[...]
            cp_o = pltpu.make_async_copy(vo, out_ref.at[rows, :], dma_sems.at[2])
            cp_o.start()
            cp_o.wait()

    out, _recv = pl.pallas_call(
        body,
        out_shape=[
            jax.ShapeDtypeStruct((M, HALF), jnp.float32),
            jax.ShapeDtypeStruct((M, HALF), jnp.float32),
        ],
        in_specs=[pl.BlockSpec(memory_space=pl.ANY)],
        out_specs=[
            pl.BlockSpec(memory_space=pl.ANY),
            pl.BlockSpec(memory_space=pl.ANY),
        ],
        scratch_shapes=[
            pltpu.VMEM((CHUNK, HALF), jnp.float32),
            pltpu.VMEM((CHUNK, HALF), jnp.float32),
            pltpu.VMEM((CHUNK, HALF), jnp.float32),
            pltpu.SemaphoreType.DMA((3,)),
            pltpu.SemaphoreType.DMA,
            pltpu.SemaphoreType.DMA,
        ],
        compiler_params=pltpu.CompilerParams(collective_id=0),
    )(x)
    return out


# device time: 394216 ns/iter; 2.1494x vs baseline; 2.1494x over previous
import jax
import jax.numpy as jnp
from jax import lax
from jax.experimental import pallas as pl
from jax.experimental.pallas import tpu as pltpu

M = 16384
N = 2048
HALF = N // 2
CHUNK = 2048
N_CHUNKS = M // CHUNK


def kernel(x):

    def body(
        x_ref, out_ref, recv_ref,
        vf, vs, vr, vo,
        f_sems, r_sems, out_sem, send_sems, recv_sems,
    ):
        my_x = lax.axis_index("x")
        my_y = lax.axis_index("y")
        my_z = lax.axis_index("z")
        peer_z = 1 - my_z
        peer = (my_x, my_y, peer_z)

        barrier_sem = pltpu.get_barrier_semaphore()
        pl.semaphore_signal(
            barrier_sem, inc=1, device_id=peer,
            device_id_type=pl.DeviceIdType.MESH,
        )
        pl.semaphore_wait(barrier_sem, 1)

        send_cols = pl.ds(peer_z * HALF, HALF)
        my_cols = pl.ds(my_z * HALF, HALF)

        def rdma_for(i, slot):
            return pltpu.make_async_remote_copy(
                src_ref=vs.at[slot],
                dst_ref=recv_ref.at[pl.ds(i * CHUNK, CHUNK), :],
                send_sem=send_sems.at[i],
                recv_sem=recv_sems.at[i],
                device_id=peer,
                device_id_type=pl.DeviceIdType.MESH,
            )

        def load_f(i, slot, cols):
            rows = pl.ds(i * CHUNK, CHUNK)
            return pltpu.make_async_copy(
                x_ref.at[0, rows, cols], vf.at[slot], f_sems.at[slot]
            )

        load_f(0, 0, send_cols).start()
        for i in range(N_CHUNKS):
            slot = i % 2
            if i + 1 < N_CHUNKS:
                load_f(i + 1, (i + 1) % 2, send_cols).start()
            load_f(i, slot, send_cols).wait()
            if i >= 2:
                rdma_for(i - 2, slot).wait_send()
            vs[slot] = vf[slot].astype(jnp.bfloat16)
            rdma_for(i, slot).start()

        load_f(0, 0, my_cols).start()
        for i in range(N_CHUNKS):
            slot = i % 2
            rows = pl.ds(i * CHUNK, CHUNK)
            if i + 1 < N_CHUNKS:
                load_f(i + 1, (i + 1) % 2, my_cols).start()
            rdma_for(i, 0).wait_recv()
            cr = pltpu.make_async_copy(
                recv_ref.at[rows, :], vr.at[slot], r_sems.at[slot]
            )
            cr.start()
            load_f(i, slot, my_cols).wait()
            cr.wait()
            if i > 0:
                pltpu.make_async_copy(
                    vo.at[(i - 1) % 2],
                    out_ref.at[pl.ds((i - 1) * CHUNK, CHUNK), :],
                    out_sem,
                ).wait()
            vo[slot] = (vf[slot] + vr[slot].astype(jnp.float32)).astype(
                jnp.bfloat16
            )
            pltpu.make_async_copy(vo.at[slot], out_ref.at[rows, :], out_sem).start()
        pltpu.make_async_copy(
            vo.at[(N_CHUNKS - 1) % 2],
            out_ref.at[pl.ds((N_CHUNKS - 1) * CHUNK, CHUNK), :],
            out_sem,
        ).wait()

        rdma_for(N_CHUNKS - 2, (N_CHUNKS - 2) % 2).wait_send()
        rdma_for(N_CHUNKS - 1, (N_CHUNKS - 1) % 2).wait_send()

    out, _recv = pl.pallas_call(
        body,
        out_shape=[
            jax.ShapeDtypeStruct((M, HALF), jnp.bfloat16),
            jax.ShapeDtypeStruct((M, HALF), jnp.bfloat16),
        ],
        in_specs=[pl.BlockSpec(memory_space=pl.ANY)],
        out_specs=[
            pl.BlockSpec(memory_space=pl.ANY),
            pl.BlockSpec(memory_space=pl.ANY),
        ],
        scratch_shapes=[
            pltpu.VMEM((2, CHUNK, HALF), jnp.float32),
            pltpu.VMEM((2, CHUNK, HALF), jnp.bfloat16),
            pltpu.VMEM((2, CHUNK, HALF), jnp.bfloat16),
            pltpu.VMEM((2, CHUNK, HALF), jnp.bfloat16),
            pltpu.SemaphoreType.DMA((2,)),
            pltpu.SemaphoreType.DMA((2,)),
            pltpu.SemaphoreType.DMA,
            pltpu.SemaphoreType.DMA((N_CHUNKS,)),
            pltpu.SemaphoreType.DMA((N_CHUNKS,)),
        ],
        compiler_params=pltpu.CompilerParams(
            collective_id=0, vmem_limit_bytes=56 * 1024 * 1024
        ),
    )(x)
    return out


# device time: 215743 ns/iter; 3.9275x vs baseline; 1.8272x over previous
import jax
import jax.numpy as jnp
from jax import lax
from jax.experimental import pallas as pl
from jax.experimental.pallas import tpu as pltpu

M = 16384
N = 2048
HALF = N // 2
QROWS = M // 4
CHUNK = QROWS // 2


def kernel(x):

    def body(
        x_ref, out_ref,
        vf, vs, rz, vpart,
        f_sems, po_sems,
        zs_sems, zr_sems,
        r1xs, r1xr, r1ys, r1yr,
        r2xs, r2xr, r2ys, r2yr,
    ):
        my_x = lax.axis_index("x")
        my_y = lax.axis_index("y")
        my_z = lax.axis_index("z")
        zp = (my_x, my_y, 1 - my_z)
        xn = (1 - my_x, my_y, my_z)
        yn = (my_x, 1 - my_y, my_z)

        q_me = 2 * my_x + my_y
        q_xn = 2 * (1 - my_x) + my_y
        q_yn = 2 * my_x + (1 - my_y)
        q_dg = 2 * (1 - my_x) + (1 - my_y)
        row_me = q_me * QROWS
        row_xn = q_xn * QROWS
        row_yn = q_yn * QROWS
        row_dg = q_dg * QROWS
        hx = lax.rem(my_x + my_y, 2)
        hy = 1 - hx

        send_cols = pl.ds((1 - my_z) * HALF, HALF)
        my_cols = pl.ds(my_z * HALF, HALF)

        barrier_sem = pltpu.get_barrier_semaphore()
        for nbr in (zp, xn, yn):
            pl.semaphore_signal(
                barrier_sem, inc=1, device_id=nbr,
                device_id_type=pl.DeviceIdType.MESH,
            )
        pl.semaphore_wait(barrier_sem, 3)

        def zrdma(c):
            return pltpu.make_async_remote_copy(
                src_ref=vs.at[c],
                dst_ref=rz.at[c],
                send_sem=zs_sems.at[c],
                recv_sem=zr_sems.at[c],
                device_id=zp,
                device_id_type=pl.DeviceIdType.MESH,
            )

        def out_rows(base, c):
            return pl.ds(base + c * CHUNK, CHUNK)

        def load_f(c, cols):
            return pltpu.make_async_copy(
                x_ref.at[0, out_rows(row_me, c), cols],
                vf.at[c],
                f_sems.at[c],
            )

        load_f(0, send_cols).start()
        load_f(1, send_cols).start()
        for c in range(2):
            load_f(c, send_cols).wait()
            vs[c] = vf[c].astype(jnp.bfloat16)
            zrdma(c).start()

        def r1(c, dev, s_sems, r_sems):
            return pltpu.make_async_remote_copy(
                src_ref=vpart.at[c],
                dst_ref=out_ref.at[out_rows(row_me, c), :],
                send_sem=s_sems.at[c],
                recv_sem=r_sems.at[c],
                device_id=dev,
                device_id_type=pl.DeviceIdType.MESH,
            )

        load_f(0, my_cols).start()
        for c in range(2):
            if c == 0:
                load_f(1, my_cols).start()
            load_f(c, my_cols).wait()
            zrdma(c).wait_recv()
            vpart[c] = (vf[c] + rz[c].astype(jnp.float32)).astype(jnp.bfloat16)
            pltpu.make_async_copy(
                vpart.at[c], out_ref.at[out_rows(row_me, c), :], po_sems.at[c]
            ).start()
            r1(c, xn, r1xs, r1xr).start()
            r1(c, yn, r1ys, r1yr).start()

        def r1_recv(c, base, r_sems):
            return pltpu.make_async_remote_copy(
                src_ref=vpart.at[c],
                dst_ref=out_ref.at[out_rows(base, c), :],
                send_sem=r1xs.at[c],
                recv_sem=r_sems.at[c],
                device_id=xn,
                device_id_type=pl.DeviceIdType.MESH,
            )

        for c in range(2):
            r1_recv(c, row_yn, r1yr).wait_recv()
        fwd_x = pltpu.make_async_remote_copy(
            src_ref=out_ref.at[pl.ds(row_yn + hx * CHUNK, CHUNK), :],
            dst_ref=out_ref.at[pl.ds(row_yn + hx * CHUNK, CHUNK), :],
            send_sem=r2xs,
            recv_sem=r2xr,
            device_id=xn,
            device_id_type=pl.DeviceIdType.MESH,
        )
        fwd_x.start()

        for c in range(2):
            r1_recv(c, row_xn, r1xr).wait_recv()
        fwd_y = pltpu.make_async_remote_copy(
            src_ref=out_ref.at[pl.ds(row_xn + hy * CHUNK, CHUNK), :],
            dst_ref=out_ref.at[pl.ds(row_xn + hy * CHUNK, CHUNK), :],
            send_sem=r2ys,
            recv_sem=r2yr,
            device_id=yn,
            device_id_type=pl.DeviceIdType.MESH,
        )
        fwd_y.start()

        pltpu.make_async_remote_copy(
            src_ref=out_ref.at[pl.ds(row_dg + hy * CHUNK, CHUNK), :],
            dst_ref=out_ref.at[pl.ds(row_dg + hy * CHUNK, CHUNK), :],
            send_sem=r2xs,
            recv_sem=r2xr,
            device_id=xn,
            device_id_type=pl.DeviceIdType.MESH,
        ).wait_recv()
        pltpu.make_async_remote_copy(
            src_ref=out_ref.at[pl.ds(row_dg + hx * CHUNK, CHUNK), :],
            dst_ref=out_ref.at[pl.ds(row_dg + hx * CHUNK, CHUNK), :],
            send_sem=r2ys,
            recv_sem=r2yr,
            device_id=yn,
            device_id_type=pl.DeviceIdType.MESH,
        ).wait_recv()

        for c in range(2):
            pltpu.make_async_copy(
                vpart.at[c], out_ref.at[out_rows(row_me, c), :], po_sems.at[c]
            ).wait()
            zrdma(c).wait_send()
            r1(c, xn, r1xs, r1xr).wait_send()
            r1(c, yn, r1ys, r1yr).wait_send()
        fwd_x.wait_send()
        fwd_y.wait_send()

    out = pl.pallas_call(
        body,
        out_shape=jax.ShapeDtypeStruct((M, HALF), jnp.bfloat16),
        in_specs=[pl.BlockSpec(memory_space=pl.ANY)],
        out_specs=pl.BlockSpec(memory_space=pl.ANY),
        scratch_shapes=[
            pltpu.VMEM((2, CHUNK, HALF), jnp.float32),
            pltpu.VMEM((2, CHUNK, HALF), jnp.bfloat16),
            pltpu.VMEM((2, CHUNK, HALF), jnp.bfloat16),
            pltpu.VMEM((2, CHUNK, HALF), jnp.bfloat16),
            pltpu.SemaphoreType.DMA((2,)),
            pltpu.SemaphoreType.DMA((2,)),
            pltpu.SemaphoreType.DMA((2,)),
            pltpu.SemaphoreType.DMA((2,)),
            pltpu.SemaphoreType.DMA((2,)),
            pltpu.SemaphoreType.DMA((2,)),
            pltpu.SemaphoreType.DMA((2,)),
            pltpu.SemaphoreType.DMA((2,)),
            pltpu.SemaphoreType.DMA,
            pltpu.SemaphoreType.DMA,
            pltpu.SemaphoreType.DMA,
            pltpu.SemaphoreType.DMA,
        ],
        compiler_params=pltpu.CompilerParams(
            collective_id=0, vmem_limit_bytes=56 * 1024 * 1024
        ),
    )(x)
    return out


# device time: 191158 ns/iter; 4.4326x vs baseline; 1.1286x over previous
import jax
import jax.numpy as jnp
from jax import lax
from jax.experimental import pallas as pl
from jax.experimental.pallas import tpu as pltpu

M = 16384
N = 2048
HALF = N // 2
QROWS = M // 4
NC = 4
CHUNK = QROWS // NC
FROWS = QROWS // 2


def kernel(x):

    def body(
        x_ref, out_ref,
        vf, vs, rz, vpart,
        f_sems, po_sems,
        zs_sems, zr_sems,
        r1xs, r1xr, r1ys, r1yr,
        r2xs, r2xr, r2ys, r2yr,
    ):
        my_x = lax.axis_index("x")
        my_y = lax.axis_index("y")
        my_z = lax.axis_index("z")
        zp = (my_x, my_y, 1 - my_z)
        xn = (1 - my_x, my_y, my_z)
        yn = (my_x, 1 - my_y, my_z)

        q_me = 2 * my_x + my_y
        q_xn = 2 * (1 - my_x) + my_y
        q_yn = 2 * my_x + (1 - my_y)
        q_dg = 2 * (1 - my_x) + (1 - my_y)
        row_me = q_me * QROWS
        row_xn = q_xn * QROWS
        row_yn = q_yn * QROWS
        row_dg = q_dg * QROWS
        hx = lax.rem(my_x + my_y, 2)
        hy = 1 - hx

        send_cols = pl.ds((1 - my_z) * HALF, HALF)
        my_cols = pl.ds(my_z * HALF, HALF)

        barrier_sem = pltpu.get_barrier_semaphore()
        for nbr in (zp, xn, yn):
            pl.semaphore_signal(
                barrier_sem, inc=1, device_id=nbr,
                device_id_type=pl.DeviceIdType.MESH,
            )
        pl.semaphore_wait(barrier_sem, 3)

        def zrdma(c):
            return pltpu.make_async_remote_copy(
                src_ref=vs.at[c],
                dst_ref=rz.at[c],
                send_sem=zs_sems.at[c],
                recv_sem=zr_sems.at[c],
                device_id=zp,
                device_id_type=pl.DeviceIdType.MESH,
            )

        def rows_of(base, c):
            return pl.ds(base + c * CHUNK, CHUNK)

        def load_f(c, cols):
            return pltpu.make_async_copy(
                x_ref.at[0, rows_of(row_me, c), cols],
                vf.at[c % 2],
                f_sems.at[c % 2],
            )

        load_f(0, send_cols).start()
        for c in range(NC):
            if c + 1 < NC:
                load_f(c + 1, send_cols).start()
            load_f(c, send_cols).wait()
            vs[c] = vf[c % 2].astype(jnp.bfloat16)
            zrdma(c).start()

        def r1(c, dev, s_sems, r_sems):
            return pltpu.make_async_remote_copy(
                src_ref=vpart.at[c],
                dst_ref=out_ref.at[rows_of(row_me, c), :],
                send_sem=s_sems.at[c],
                recv_sem=r_sems.at[c],
                device_id=dev,
                device_id_type=pl.DeviceIdType.MESH,
            )

        load_f(0, my_cols).start()
        for c in range(NC):
            if c + 1 < NC:
                load_f(c + 1, my_cols).start()
            load_f(c, my_cols).wait()
            zrdma(c).wait_recv()
            vpart[c] = (vf[c % 2] + rz[c].astype(jnp.float32)).astype(
                jnp.bfloat16
            )
            pltpu.make_async_copy(
                vpart.at[c], out_ref.at[rows_of(row_me, c), :], po_sems.at[c]
            ).start()
            r1(c, xn, r1xs, r1xr).start()
            r1(c, yn, r1ys, r1yr).start()

        def r1_recv(c, base, r_sems):
            return pltpu.make_async_remote_copy(
                src_ref=vpart.at[c],
                dst_ref=out_ref.at[rows_of(base, c), :],
                send_sem=r1xs.at[c],
                recv_sem=r_sems.at[c],
                device_id=xn,
                device_id_type=pl.DeviceIdType.MESH,
            )

        def fwd(base, h, dev, s_sem, r_sem):
            rows = pl.ds(base + h * FROWS, FROWS)
            return pltpu.make_async_remote_copy(
                src_ref=out_ref.at[rows, :],
                dst_ref=out_ref.at[rows, :],
                send_sem=s_sem,
                recv_sem=r_sem,
                device_id=dev,
                device_id_type=pl.DeviceIdType.MESH,
            )

        @pl.when(hx == 0)
        def _():
            r1_recv(0, row_yn, r1yr).wait_recv()
            r1_recv(1, row_yn, r1yr).wait_recv()

        @pl.when(hx == 1)
        def _():
            r1_recv(2, row_yn, r1yr).wait_recv()
            r1_recv(3, row_yn, r1yr).wait_recv()

        fwd(row_yn, hx, xn, r2xs, r2xr).start()

        @pl.when(hy == 0)
        def _():
            r1_recv(0, row_xn, r1xr).wait_recv()
            r1_recv(1, row_xn, r1xr).wait_recv()

        @pl.when(hy == 1)
        def _():
            r1_recv(2, row_xn, r1xr).wait_recv()
            r1_recv(3, row_xn, r1xr).wait_recv()

        fwd(row_xn, hy, yn, r2ys, r2yr).start()

        @pl.when(hx == 0)
        def _():
            r1_recv(2, row_yn, r1yr).wait_recv()
            r1_recv(3, row_yn, r1yr).wait_recv()

        @pl.when(hx == 1)
        def _():
            r1_recv(0, row_yn, r1yr).wait_recv()
            r1_recv(1, row_yn, r1yr).wait_recv()

        @pl.when(hy == 0)
        def _():
            r1_recv(2, row_xn, r1xr).wait_recv()
            r1_recv(3, row_xn, r1xr).wait_recv()

        @pl.when(hy == 1)
        def _():
            r1_recv(0, row_xn, r1xr).wait_recv()
            r1_recv(1, row_xn, r1xr).wait_recv()

        fwd(row_dg, hy, xn, r2xs, r2xr).wait_recv()
        fwd(row_dg, hx, yn, r2ys, r2yr).wait_recv()

        for c in range(NC):
            pltpu.make_async_copy(
                vpart.at[c], out_ref.at[rows_of(row_me, c), :], po_sems.at[c]
            ).wait()
            zrdma(c).wait_send()
            r1(c, xn, r1xs, r1xr).wait_send()
            r1(c, yn, r1ys, r1yr).wait_send()
        fwd(row_yn, hx, xn, r2xs, r2xr).wait_send()
        fwd(row_xn, hy, yn, r2ys, r2yr).wait_send()

    out = pl.pallas_call(
        body,
        out_shape=jax.ShapeDtypeStruct((M, HALF), jnp.bfloat16),
        in_specs=[pl.BlockSpec(memory_space=pl.ANY)],
        out_specs=pl.BlockSpec(memory_space=pl.ANY),
        scratch_shapes=[
            pltpu.VMEM((2, CHUNK, HALF), jnp.float32),
            pltpu.VMEM((NC, CHUNK, HALF), jnp.bfloat16),
            pltpu.VMEM((NC, CHUNK, HALF), jnp.bfloat16),
            pltpu.VMEM((NC, CHUNK, HALF), jnp.bfloat16),
            pltpu.SemaphoreType.DMA((2,)),
            pltpu.SemaphoreType.DMA((NC,)),
            pltpu.SemaphoreType.DMA((NC,)),
            pltpu.SemaphoreType.DMA((NC,)),
            pltpu.SemaphoreType.DMA((NC,)),
            pltpu.SemaphoreType.DMA((NC,)),
            pltpu.SemaphoreType.DMA((NC,)),
            pltpu.SemaphoreType.DMA((NC,)),
            pltpu.SemaphoreType.DMA,
            pltpu.SemaphoreType.DMA,
            pltpu.SemaphoreType.DMA,
            pltpu.SemaphoreType.DMA,
        ],
        compiler_params=pltpu.CompilerParams(
            collective_id=0, vmem_limit_bytes=56 * 1024 * 1024
        ),
    )(x)
    return out


# device time: 179691 ns/iter; 4.7155x vs baseline; 1.0638x over previous
import jax
import jax.numpy as jnp
from jax import lax
from jax.experimental import pallas as pl
from jax.experimental.pallas import tpu as pltpu

M = 16384
N = 2048
HALF = N // 2
QROWS = M // 4
NC = 8
CHUNK = QROWS // NC
FROWS = QROWS // 2


def kernel(x):

    def body(
        x_ref, out_ref,
        vf, vs, rz, vpart,
        f_sems, po_sems,
        zs_sems, zr_sems,
        r1xs, r1xr, r1ys, r1yr,
        r2xs, r2xr, r2ys, r2yr,
    ):
        my_x = lax.axis_index("x")
        my_y = lax.axis_index("y")
        my_z = lax.axis_index("z")
        zp = (my_x, my_y, 1 - my_z)
        xn = (1 - my_x, my_y, my_z)
        yn = (my_x, 1 - my_y, my_z)

        q_me = 2 * my_x + my_y
        q_xn = 2 * (1 - my_x) + my_y
        q_yn = 2 * my_x + (1 - my_y)
        q_dg = 2 * (1 - my_x) + (1 - my_y)
        row_me = q_me * QROWS
        row_xn = q_xn * QROWS
        row_yn = q_yn * QROWS
        row_dg = q_dg * QROWS
        hx = lax.rem(my_x + my_y, 2)
        hy = 1 - hx

        send_cols = pl.ds((1 - my_z) * HALF, HALF)
        my_cols = pl.ds(my_z * HALF, HALF)

        barrier_sem = pltpu.get_barrier_semaphore()
        for nbr in (zp, xn, yn):
            pl.semaphore_signal(
                barrier_sem, inc=1, device_id=nbr,
                device_id_type=pl.DeviceIdType.MESH,
            )
        pl.semaphore_wait(barrier_sem, 3)

        def zrdma(c):
            return pltpu.make_async_remote_copy(
                src_ref=vs.at[c],
                dst_ref=rz.at[c],
                send_sem=zs_sems.at[c],
                recv_sem=zr_sems.at[c],
                device_id=zp,
                device_id_type=pl.DeviceIdType.MESH,
            )

        def rows_of(base, c):
            return pl.ds(base + c * CHUNK, CHUNK)

        def load_f(c, cols):
            return pltpu.make_async_copy(
                x_ref.at[0, rows_of(row_me, c), cols],
                vf.at[c % 2],
                f_sems.at[c % 2],
            )

        load_f(0, send_cols).start()
        for c in range(NC):
            if c + 1 < NC:
                load_f(c + 1, send_cols).start()
            load_f(c, send_cols).wait()
            vs[c] = vf[c % 2].astype(jnp.bfloat16)
            zrdma(c).start()

        def r1(c, dev, s_sems, r_sems):
            return pltpu.make_async_remote_copy(
                src_ref=vpart.at[c],
                dst_ref=out_ref.at[rows_of(row_me, c), :],
                send_sem=s_sems.at[c],
                recv_sem=r_sems.at[c],
                device_id=dev,
                device_id_type=pl.DeviceIdType.MESH,
            )

        load_f(0, my_cols).start()
        for c in range(NC):
            if c + 1 < NC:
                load_f(c + 1, my_cols).start()
            load_f(c, my_cols).wait()
            zrdma(c).wait_recv()
            vpart[c] = (vf[c % 2] + rz[c].astype(jnp.float32)).astype(
                jnp.bfloat16
            )
            r1(c, xn, r1xs, r1xr).start()
            r1(c, yn, r1ys, r1yr).start()
            pltpu.make_async_copy(
                vpart.at[c], out_ref.at[rows_of(row_me, c), :], po_sems.at[c]
            ).start()

        def r1_recv(c, base, r_sems):
            return pltpu.make_async_remote_copy(
                src_ref=vpart.at[c],
                dst_ref=out_ref.at[rows_of(base, c), :],
                send_sem=r1xs.at[c],
                recv_sem=r_sems.at[c],
                device_id=xn,
                device_id_type=pl.DeviceIdType.MESH,
            )

        def fwd(base, h, dev, s_sem, r_sem):
            rows = pl.ds(base + h * FROWS, FROWS)
            return pltpu.make_async_remote_copy(
                src_ref=out_ref.at[rows, :],
                dst_ref=out_ref.at[rows, :],
                send_sem=s_sem,
                recv_sem=r_sem,
                device_id=dev,
                device_id_type=pl.DeviceIdType.MESH,
            )

        HC = NC // 2

        @pl.when(hx == 0)
        def _():
            for c in range(HC):
                r1_recv(c, row_yn, r1yr).wait_recv()

        @pl.when(hx == 1)
        def _():
            for c in range(HC, NC):
                r1_recv(c, row_yn, r1yr).wait_recv()

        fwd(row_yn, hx, xn, r2xs, r2xr).start()

        @pl.when(hy == 0)
        def _():
            for c in range(HC):
                r1_recv(c, row_xn, r1xr).wait_recv()

        @pl.when(hy == 1)
        def _():
            for c in range(HC, NC):
                r1_recv(c, row_xn, r1xr).wait_recv()

        fwd(row_xn, hy, yn, r2ys, r2yr).start()

        @pl.when(hx == 0)
        def _():
            for c in range(HC, NC):
                r1_recv(c, row_yn, r1yr).wait_recv()

        @pl.when(hx == 1)
        def _():
            for c in range(HC):
                r1_recv(c, row_yn, r1yr).wait_recv()

        @pl.when(hy == 0)
        def _():
            for c in range(HC, NC):
                r1_recv(c, row_xn, r1xr).wait_recv()

        @pl.when(hy == 1)
        def _():
            for c in range(HC):
                r1_recv(c, row_xn, r1xr).wait_recv()

        fwd(row_dg, hy, xn, r2xs, r2xr).wait_recv()
        fwd(row_dg, hx, yn, r2ys, r2yr).wait_recv()

        for c in range(NC):
            pltpu.make_async_copy(
                vpart.at[c], out_ref.at[rows_of(row_me, c), :], po_sems.at[c]
            ).wait()
            zrdma(c).wait_send()
            r1(c, xn, r1xs, r1xr).wait_send()
            r1(c, yn, r1ys, r1yr).wait_send()
        fwd(row_yn, hx, xn, r2xs, r2xr).wait_send()
        fwd(row_xn, hy, yn, r2ys, r2yr).wait_send()

    out = pl.pallas_call(
        body,
        out_shape=jax.ShapeDtypeStruct((M, HALF), jnp.bfloat16),
        in_specs=[pl.BlockSpec(memory_space=pl.ANY)],
        out_specs=pl.BlockSpec(memory_space=pl.ANY),
        scratch_shapes=[
            pltpu.VMEM((2, CHUNK, HALF), jnp.float32),
            pltpu.VMEM((NC, CHUNK, HALF), jnp.bfloat16),
            pltpu.VMEM((NC, CHUNK, HALF), jnp.bfloat16),
            pltpu.VMEM((NC, CHUNK, HALF), jnp.bfloat16),
            pltpu.SemaphoreType.DMA((2,)),
            pltpu.SemaphoreType.DMA((NC,)),
            pltpu.SemaphoreType.DMA((NC,)),
            pltpu.SemaphoreType.DMA((NC,)),
            pltpu.SemaphoreType.DMA((NC,)),
            pltpu.SemaphoreType.DMA((NC,)),
            pltpu.SemaphoreType.DMA((NC,)),
            pltpu.SemaphoreType.DMA((NC,)),
            pltpu.SemaphoreType.DMA,
            pltpu.SemaphoreType.DMA,
            pltpu.SemaphoreType.DMA,
            pltpu.SemaphoreType.DMA,
        ],
        compiler_params=pltpu.CompilerParams(
            collective_id=0, vmem_limit_bytes=56 * 1024 * 1024
        ),
    )(x)
    return out


# device time: 178391 ns/iter; 4.7499x vs baseline; 1.0073x over previous
import jax
import jax.numpy as jnp
from jax import lax
from jax.experimental import pallas as pl
from jax.experimental.pallas import tpu as pltpu

M = 16384
N = 2048
HALF = N // 2
QROWS = M // 4
NC = 16
CHUNK = QROWS // NC
FROWS = QROWS // 2


def kernel(x):

    def body(
        x_ref, out_ref,
        vf, vs, rz, vpart,
        f_sems, po_sems,
        zs_sems, zr_sems,
        r1xs, r1xr, r1ys, r1yr,
        r2xs, r2xr, r2ys, r2yr,
    ):
        my_x = lax.axis_index("x")
        my_y = lax.axis_index("y")
        my_z = lax.axis_index("z")
        zp = (my_x, my_y, 1 - my_z)
        xn = (1 - my_x, my_y, my_z)
        yn = (my_x, 1 - my_y, my_z)

        q_me = 2 * my_x + my_y
        q_xn = 2 * (1 - my_x) + my_y
        q_yn = 2 * my_x + (1 - my_y)
        q_dg = 2 * (1 - my_x) + (1 - my_y)
        row_me = q_me * QROWS
        row_xn = q_xn * QROWS
        row_yn = q_yn * QROWS
        row_dg = q_dg * QROWS
        hx = lax.rem(my_x + my_y, 2)
        hy = 1 - hx

        send_cols = pl.ds((1 - my_z) * HALF, HALF)
        my_cols = pl.ds(my_z * HALF, HALF)

        barrier_sem = pltpu.get_barrier_semaphore()
        for nbr in (zp, xn, yn):
            pl.semaphore_signal(
                barrier_sem, inc=1, device_id=nbr,
                device_id_type=pl.DeviceIdType.MESH,
            )
        pl.semaphore_wait(barrier_sem, 3)

        def zrdma(c):
            return pltpu.make_async_remote_copy(
                src_ref=vs.at[c],
                dst_ref=rz.at[c],
                send_sem=zs_sems.at[c],
                recv_sem=zr_sems.at[c],
                device_id=zp,
                device_id_type=pl.DeviceIdType.MESH,
            )

        def rows_of(base, c):
            return pl.ds(base + c * CHUNK, CHUNK)

        def load_f(c, cols):
            return pltpu.make_async_copy(
                x_ref.at[0, rows_of(row_me, c), cols],
                vf.at[c % 2],
                f_sems.at[c % 2],
            )

        load_f(0, send_cols).start()
        for c in range(NC):
            if c + 1 < NC:
                load_f(c + 1, send_cols).start()
            load_f(c, send_cols).wait()
            vs[c] = vf[c % 2].astype(jnp.bfloat16)
            zrdma(c).start()

        def r1(c, dev, s_sems, r_sems):
            return pltpu.make_async_remote_copy(
                src_ref=vpart.at[c],
                dst_ref=out_ref.at[rows_of(row_me, c), :],
                send_sem=s_sems.at[c],
                recv_sem=r_sems.at[c],
                device_id=dev,
                device_id_type=pl.DeviceIdType.MESH,
            )

        load_f(0, my_cols).start()
        for c in range(NC):
            if c + 1 < NC:
                load_f(c + 1, my_cols).start()
            load_f(c, my_cols).wait()
            zrdma(c).wait_recv()
            vpart[c] = (vf[c % 2] + rz[c].astype(jnp.float32)).astype(
                jnp.bfloat16
            )
            r1(c, xn, r1xs, r1xr).start()
            r1(c, yn, r1ys, r1yr).start()
            pltpu.make_async_copy(
                vpart.at[c], out_ref.at[rows_of(row_me, c), :], po_sems.at[c]
            ).start()

        def r1_recv(c, base, r_sems):
            return pltpu.make_async_remote_copy(
                src_ref=vpart.at[c],
                dst_ref=out_ref.at[rows_of(base, c), :],
                send_sem=r1xs.at[c],
                recv_sem=r_sems.at[c],
                device_id=xn,
                device_id_type=pl.DeviceIdType.MESH,
            )

        def fwd(base, h, dev, s_sem, r_sem):
            rows = pl.ds(base + h * FROWS, FROWS)
            return pltpu.make_async_remote_copy(
                src_ref=out_ref.at[rows, :],
                dst_ref=out_ref.at[rows, :],
                send_sem=s_sem,
                recv_sem=r_sem,
                device_id=dev,
                device_id_type=pl.DeviceIdType.MESH,
            )

        HC = NC // 2

        @pl.when(hx == 0)
        def _():
            for c in range(HC):
                r1_recv(c, row_yn, r1yr).wait_recv()

        @pl.when(hx == 1)
        def _():
            for c in range(HC, NC):
                r1_recv(c, row_yn, r1yr).wait_recv()

        fwd(row_yn, hx, xn, r2xs, r2xr).start()

        @pl.when(hy == 0)
        def _():
            for c in range(HC):
                r1_recv(c, row_xn, r1xr).wait_recv()

        @pl.when(hy == 1)
        def _():
            for c in range(HC, NC):
                r1_recv(c, row_xn, r1xr).wait_recv()

        fwd(row_xn, hy, yn, r2ys, r2yr).start()

        @pl.when(hx == 0)
        def _():
            for c in range(HC, NC):
                r1_recv(c, row_yn, r1yr).wait_recv()

        @pl.when(hx == 1)
        def _():
            for c in range(HC):
                r1_recv(c, row_yn, r1yr).wait_recv()

        @pl.when(hy == 0)
        def _():
            for c in range(HC, NC):
                r1_recv(c, row_xn, r1xr).wait_recv()

        @pl.when(hy == 1)
        def _():
            for c in range(HC):
                r1_recv(c, row_xn, r1xr).wait_recv()

        fwd(row_dg, hy, xn, r2xs, r2xr).wait_recv()
        fwd(row_dg, hx, yn, r2ys, r2yr).wait_recv()

        for c in range(NC):
            pltpu.make_async_copy(
                vpart.at[c], out_ref.at[rows_of(row_me, c), :], po_sems.at[c]
            ).wait()
            zrdma(c).wait_send()
            r1(c, xn, r1xs, r1xr).wait_send()
            r1(c, yn, r1ys, r1yr).wait_send()
        fwd(row_yn, hx, xn, r2xs, r2xr).wait_send()
        fwd(row_xn, hy, yn, r2ys, r2yr).wait_send()

    out = pl.pallas_call(
        body,
        out_shape=jax.ShapeDtypeStruct((M, HALF), jnp.bfloat16),
        in_specs=[pl.BlockSpec(memory_space=pl.ANY)],
        out_specs=pl.BlockSpec(memory_space=pl.ANY),
        scratch_shapes=[
            pltpu.VMEM((2, CHUNK, HALF), jnp.float32),
            pltpu.VMEM((NC, CHUNK, HALF), jnp.bfloat16),
            pltpu.VMEM((NC, CHUNK, HALF), jnp.bfloat16),
            pltpu.VMEM((NC, CHUNK, HALF), jnp.bfloat16),
            pltpu.SemaphoreType.DMA((2,)),
            pltpu.SemaphoreType.DMA((NC,)),
            pltpu.SemaphoreType.DMA((NC,)),
            pltpu.SemaphoreType.DMA((NC,)),
            pltpu.SemaphoreType.DMA((NC,)),
            pltpu.SemaphoreType.DMA((NC,)),
            pltpu.SemaphoreType.DMA((NC,)),
            pltpu.SemaphoreType.DMA((NC,)),
            pltpu.SemaphoreType.DMA,
            pltpu.SemaphoreType.DMA,
            pltpu.SemaphoreType.DMA,
            pltpu.SemaphoreType.DMA,
        ],
        compiler_params=pltpu.CompilerParams(
            collective_id=0, vmem_limit_bytes=56 * 1024 * 1024
        ),
    )(x)
    return out
